# Initial kernel scaffold; baseline (speedup 1.0000x reference)
import jax
import jax.numpy as jnp
from jax.experimental import pallas as pl


def kernel(x, edge_index, W1, b1, W2, b2):
    raise NotImplementedError("write your pallas kernel here")


# speedup vs baseline: 17.3071x; 17.3071x over previous
import functools

import jax
import jax.numpy as jnp
from jax import lax
from jax.experimental import pallas as pl
from jax.experimental.pallas import tpu as pltpu
from jax.experimental.pallas import tpu_sc as plsc

N = 10000
E = 320000
D_IN = 128
D_HID = 64
D_OUT = 16

NW = 32
CHUNK = 128
CHUNKS_PER_W = 79
EPAD = NW * CHUNK * CHUNKS_PER_W
NROWS = 10240
ROWS_PER_TILE = NROWS // 16
PAD_DST = N + 200
DEG_W = 16

_mesh = plsc.VectorSubcoreMesh(core_axis_name="c", subcore_axis_name="s")
_sc_params = pltpu.CompilerParams(use_tc_tiling_on_sc=False)


def _wid():
    return lax.axis_index("s") * 2 + lax.axis_index("c")


@functools.partial(
    pl.kernel,
    out_type=jax.ShapeDtypeStruct((2, NROWS, DEG_W), jnp.float32),
    mesh=_mesh,
    compiler_params=_sc_params,
    scratch_types=[
        pltpu.VMEM((CHUNK,), jnp.int32),
        pltpu.VMEM((CHUNK, DEG_W), jnp.float32),
        pltpu.VMEM_SHARED((NROWS, DEG_W), jnp.float32),
    ],
)
def _sc_degree(dst_hbm, ones_hbm, zeros_hbm, out_hbm, dst_v, ones_v, deg_sh):
    c = lax.axis_index("c")
    s = lax.axis_index("s")
    wid = _wid()
    pltpu.sync_copy(ones_hbm, ones_v)
    base = s * ROWS_PER_TILE
    pltpu.sync_copy(zeros_hbm, deg_sh.at[pl.ds(base, ROWS_PER_TILE)])
    plsc.subcore_barrier()

    def body(j, carry):
        g = wid * CHUNKS_PER_W + j
        pltpu.sync_copy(dst_hbm.at[g], dst_v)
        pltpu.sync_copy(ones_v, deg_sh.at[dst_v], add=True)
        return carry

    lax.fori_loop(0, CHUNKS_PER_W, body, 0)
    plsc.subcore_barrier()
    pltpu.sync_copy(
        deg_sh.at[pl.ds(base, ROWS_PER_TILE)],
        out_hbm.at[c, pl.ds(base, ROWS_PER_TILE)],
    )


def _make_sc_agg(d):
    @functools.partial(
        pl.kernel,
        out_type=jax.ShapeDtypeStruct((2, NROWS, d), jnp.float32),
        mesh=_mesh,
        compiler_params=_sc_params,
        scratch_types=[
            pltpu.VMEM((CHUNK,), jnp.int32),
            pltpu.VMEM((CHUNK,), jnp.int32),
            pltpu.VMEM((CHUNK, d), jnp.float32),
            pltpu.VMEM_SHARED((NROWS, d), jnp.float32),
            pltpu.SemaphoreType.DMA,
        ],
    )
    def sc_agg(src_hbm, dst_hbm, y_hbm, zeros_hbm, out_hbm,
               src_v, dst_v, rows_v, agg_sh, sem):
        c = lax.axis_index("c")
        s = lax.axis_index("s")
        wid = _wid()
        base = s * ROWS_PER_TILE
        pltpu.sync_copy(zeros_hbm, agg_sh.at[pl.ds(base, ROWS_PER_TILE)])
        plsc.subcore_barrier()

        def body(j, carry):
            g = wid * CHUNKS_PER_W + j
            pltpu.sync_copy(src_hbm.at[g], src_v)
            pltpu.sync_copy(dst_hbm.at[g], dst_v)
            pltpu.async_copy(y_hbm.at[src_v], rows_v, sem).wait()
            pltpu.sync_copy(rows_v, agg_sh.at[dst_v], add=True)
            return carry

        lax.fori_loop(0, CHUNKS_PER_W, body, 0)
        plsc.subcore_barrier()
        pltpu.sync_copy(
            agg_sh.at[pl.ds(base, ROWS_PER_TILE)],
            out_hbm.at[c, pl.ds(base, ROWS_PER_TILE)],
        )

    return sc_agg


_sc_agg64 = _make_sc_agg(D_HID)
_sc_agg16 = _make_sc_agg(D_OUT)


_R = 1024


def _dis_from(degp_ref):
    deg = degp_ref[0] + degp_ref[1] + 2.0
    return lax.rsqrt(deg)[:, 0:1]


def _tc1_body(x_ref, w1_ref, degp_ref, y1_ref):
    dis = _dis_from(degp_ref)
    xw = jnp.dot(x_ref[...], w1_ref[...], preferred_element_type=jnp.float32)
    y1_ref[...] = xw * dis


def _tc2_body(aggp_ref, y1_ref, degp_ref, w2_ref, b1_ref, y2_ref):
    dis = _dis_from(degp_ref)
    pre = (aggp_ref[0] + aggp_ref[1] + 2.0 * y1_ref[...]) * dis + b1_ref[...]
    h = jnp.maximum(pre, 0.0)
    y2_ref[...] = jnp.dot(h, w2_ref[...], preferred_element_type=jnp.float32) * dis


def _tc3_body(aggp_ref, y2_ref, degp_ref, b2_ref, out_ref):
    dis = _dis_from(degp_ref)
    o = (aggp_ref[0] + aggp_ref[1] + 2.0 * y2_ref[...]) * dis + b2_ref[...]
    m = jnp.max(o, axis=1, keepdims=True)
    e = jnp.exp(o - m)
    lse = jnp.log(jnp.sum(e, axis=1, keepdims=True))
    out_ref[...] = o - m - lse


def _row_spec(d):
    return pl.BlockSpec((_R, d), lambda i: (i, 0))


def _part_spec(d):
    return pl.BlockSpec((2, _R, d), lambda i: (0, i, 0))


def _full_spec(a, b):
    return pl.BlockSpec((a, b), lambda i: (0, 0))


_GRID = (NROWS // _R,)

_tc1 = pl.pallas_call(
    _tc1_body,
    grid=_GRID,
    in_specs=[_row_spec(D_IN), _full_spec(D_IN, D_HID), _part_spec(DEG_W)],
    out_specs=_row_spec(D_HID),
    out_shape=jax.ShapeDtypeStruct((NROWS, D_HID), jnp.float32),
)

_tc2 = pl.pallas_call(
    _tc2_body,
    grid=_GRID,
    in_specs=[_part_spec(D_HID), _row_spec(D_HID), _part_spec(DEG_W),
              _full_spec(D_HID, D_OUT), _full_spec(1, D_HID)],
    out_specs=_row_spec(D_OUT),
    out_shape=jax.ShapeDtypeStruct((NROWS, D_OUT), jnp.float32),
)

_tc3 = pl.pallas_call(
    _tc3_body,
    grid=_GRID,
    in_specs=[_part_spec(D_OUT), _row_spec(D_OUT), _part_spec(DEG_W),
              _full_spec(1, D_OUT)],
    out_specs=_row_spec(D_OUT),
    out_shape=jax.ShapeDtypeStruct((NROWS, D_OUT), jnp.float32),
)


def kernel(x, edge_index, W1, b1, W2, b2):
    ei = edge_index.astype(jnp.int32)
    npad = EPAD - E
    src = jnp.concatenate([ei[0], jnp.full((npad,), N, jnp.int32)])
    dst = jnp.concatenate([ei[1], jnp.full((npad,), PAD_DST, jnp.int32)])
    src2d = src.reshape(EPAD // CHUNK, CHUNK)
    dst2d = dst.reshape(EPAD // CHUNK, CHUNK)

    x_pad = jnp.pad(x, ((0, NROWS - N), (0, 0)))
    ones_deg = jnp.ones((CHUNK, DEG_W), jnp.float32)
    zeros_deg = jnp.zeros((ROWS_PER_TILE, DEG_W), jnp.float32)
    zeros64 = jnp.zeros((ROWS_PER_TILE, D_HID), jnp.float32)
    zeros16 = jnp.zeros((ROWS_PER_TILE, D_OUT), jnp.float32)

    degp = _sc_degree(dst2d, ones_deg, zeros_deg)
    y1 = _tc1(x_pad, W1, degp)
    agg1 = _sc_agg64(src2d, dst2d, y1, zeros64)
    y2 = _tc2(agg1, y1, degp, W2, b1.reshape(1, D_HID))
    agg2 = _sc_agg16(src2d, dst2d, y2, zeros16)
    out = _tc3(agg2, y2, degp, b2.reshape(1, D_OUT))
    return out[:N]


# speedup vs baseline: 18.7492x; 1.0833x over previous
import functools

import jax
import jax.numpy as jnp
from jax import lax
from jax.experimental import pallas as pl
from jax.experimental.pallas import tpu as pltpu
from jax.experimental.pallas import tpu_sc as plsc

N = 10000
E = 320000
D_IN = 128
D_HID = 64
D_OUT = 16

NW = 32
CHUNK = 128
CHUNKS_PER_W = 79
EPAD = NW * CHUNK * CHUNKS_PER_W
NROWS = 10240
ROWS_PER_TILE = NROWS // 16
PAD_DST = N + 200
DEG_W = 8

_mesh = plsc.VectorSubcoreMesh(core_axis_name="c", subcore_axis_name="s")
_sc_params = pltpu.CompilerParams(
    use_tc_tiling_on_sc=False, needs_layout_passes=False)


def _wid():
    return lax.axis_index("s") * 2 + lax.axis_index("c")


@functools.partial(
    pl.kernel,
    out_type=jax.ShapeDtypeStruct((2, NROWS, DEG_W), jnp.float32),
    mesh=_mesh,
    compiler_params=_sc_params,
    scratch_types=[
        pltpu.VMEM((CHUNKS_PER_W, CHUNK), jnp.int32),
        pltpu.VMEM((CHUNK, DEG_W), jnp.float32),
        pltpu.VMEM_SHARED((NROWS, DEG_W), jnp.float32),
    ],
)
def _sc_degree(dst_hbm, ones_hbm, zeros_hbm, out_hbm, dst_v, ones_v, deg_sh):
    c = lax.axis_index("c")
    s = lax.axis_index("s")
    wid = _wid()
    pltpu.sync_copy(dst_hbm.at[pl.ds(wid * CHUNKS_PER_W, CHUNKS_PER_W)], dst_v)
    pltpu.sync_copy(ones_hbm, ones_v)
    base = s * ROWS_PER_TILE
    pltpu.sync_copy(zeros_hbm, deg_sh.at[pl.ds(base, ROWS_PER_TILE)])
    plsc.subcore_barrier()

    def body(j, carry):
        pltpu.sync_copy(ones_v, deg_sh.at[dst_v.at[j]], add=True)
        return carry

    lax.fori_loop(0, CHUNKS_PER_W, body, 0)
    plsc.subcore_barrier()
    pltpu.sync_copy(
        deg_sh.at[pl.ds(base, ROWS_PER_TILE)],
        out_hbm.at[c, pl.ds(base, ROWS_PER_TILE)],
    )


def _make_sc_agg(d):
    @functools.partial(
        pl.kernel,
        out_type=jax.ShapeDtypeStruct((2, NROWS, d), jnp.float32),
        mesh=_mesh,
        compiler_params=_sc_params,
        scratch_types=[
            pltpu.VMEM((CHUNK,), jnp.int32),
            pltpu.VMEM((CHUNK,), jnp.int32),
            pltpu.VMEM((CHUNK, d), jnp.float32),
            pltpu.VMEM_SHARED((NROWS, d), jnp.float32),
            pltpu.SemaphoreType.DMA,
        ],
    )
    def sc_agg(src_hbm, dst_hbm, y_hbm, zeros_hbm, out_hbm,
               src_v, dst_v, rows_v, agg_sh, sem):
        c = lax.axis_index("c")
        s = lax.axis_index("s")
        wid = _wid()
        base = s * ROWS_PER_TILE
        pltpu.sync_copy(zeros_hbm, agg_sh.at[pl.ds(base, ROWS_PER_TILE)])
        plsc.subcore_barrier()

        def body(j, carry):
            g = wid * CHUNKS_PER_W + j
            pltpu.sync_copy(src_hbm.at[g], src_v)
            pltpu.sync_copy(dst_hbm.at[g], dst_v)
            pltpu.async_copy(y_hbm.at[src_v], rows_v, sem).wait()
            pltpu.sync_copy(rows_v, agg_sh.at[dst_v], add=True)
            return carry

        lax.fori_loop(0, CHUNKS_PER_W, body, 0)
        plsc.subcore_barrier()
        pltpu.sync_copy(
            agg_sh.at[pl.ds(base, ROWS_PER_TILE)],
            out_hbm.at[c, pl.ds(base, ROWS_PER_TILE)],
        )

    return sc_agg


_sc_agg64 = _make_sc_agg(D_HID)
_sc_agg16 = _make_sc_agg(D_OUT)


_R = 1024


def _dis_from(degp_ref):
    deg = degp_ref[0] + degp_ref[1] + 2.0
    return lax.rsqrt(deg)[:, 0:1]


def _tc1_body(x_ref, w1_ref, degp_ref, y1_ref):
    dis = _dis_from(degp_ref)
    xw = jnp.dot(x_ref[...], w1_ref[...], preferred_element_type=jnp.float32)
    y1_ref[...] = xw * dis


def _tc2_body(aggp_ref, y1_ref, degp_ref, w2_ref, b1_ref, y2_ref):
    dis = _dis_from(degp_ref)
    pre = (aggp_ref[0] + aggp_ref[1] + 2.0 * y1_ref[...]) * dis + b1_ref[...]
    h = jnp.maximum(pre, 0.0)
    y2_ref[...] = jnp.dot(h, w2_ref[...], preferred_element_type=jnp.float32) * dis


def _tc3_body(aggp_ref, y2_ref, degp_ref, b2_ref, out_ref):
    dis = _dis_from(degp_ref)
    o = (aggp_ref[0] + aggp_ref[1] + 2.0 * y2_ref[...]) * dis + b2_ref[...]
    m = jnp.max(o, axis=1, keepdims=True)
    e = jnp.exp(o - m)
    lse = jnp.log(jnp.sum(e, axis=1, keepdims=True))
    out_ref[...] = o - m - lse


def _row_spec(d):
    return pl.BlockSpec((_R, d), lambda i: (i, 0))


def _part_spec(d):
    return pl.BlockSpec((2, _R, d), lambda i: (0, i, 0))


_deg_spec = pl.BlockSpec((2, _R, DEG_W), lambda i: (0, i, 0))


def _full_spec(a, b):
    return pl.BlockSpec((a, b), lambda i: (0, 0))


_GRID = (NROWS // _R,)

_tc1 = pl.pallas_call(
    _tc1_body,
    grid=_GRID,
    in_specs=[_row_spec(D_IN), _full_spec(D_IN, D_HID), _deg_spec],
    out_specs=_row_spec(D_HID),
    out_shape=jax.ShapeDtypeStruct((NROWS, D_HID), jnp.float32),
)

_tc2 = pl.pallas_call(
    _tc2_body,
    grid=_GRID,
    in_specs=[_part_spec(D_HID), _row_spec(D_HID), _deg_spec,
              _full_spec(D_HID, D_OUT), _full_spec(1, D_HID)],
    out_specs=_row_spec(D_OUT),
    out_shape=jax.ShapeDtypeStruct((NROWS, D_OUT), jnp.float32),
)

_tc3 = pl.pallas_call(
    _tc3_body,
    grid=_GRID,
    in_specs=[_part_spec(D_OUT), _row_spec(D_OUT), _deg_spec,
              _full_spec(1, D_OUT)],
    out_specs=_row_spec(D_OUT),
    out_shape=jax.ShapeDtypeStruct((NROWS, D_OUT), jnp.float32),
)


def kernel(x, edge_index, W1, b1, W2, b2):
    ei = edge_index.astype(jnp.int32)
    npad = EPAD - E
    src = jnp.concatenate([ei[0], jnp.full((npad,), N, jnp.int32)])
    dst = jnp.concatenate([ei[1], jnp.full((npad,), PAD_DST, jnp.int32)])
    src2d = src.reshape(EPAD // CHUNK, CHUNK)
    dst2d = dst.reshape(EPAD // CHUNK, CHUNK)

    x_pad = jnp.pad(x, ((0, NROWS - N), (0, 0)))
    ones_deg = jnp.ones((CHUNK, DEG_W), jnp.float32)
    zeros_deg = jnp.zeros((ROWS_PER_TILE, DEG_W), jnp.float32)
    zeros64 = jnp.zeros((ROWS_PER_TILE, D_HID), jnp.float32)
    zeros16 = jnp.zeros((ROWS_PER_TILE, D_OUT), jnp.float32)

    degp = _sc_degree(dst2d, ones_deg, zeros_deg)
    y1 = _tc1(x_pad, W1, degp)
    agg1 = _sc_agg64(src2d, dst2d, y1, zeros64)
    y2 = _tc2(agg1, y1, degp, W2, b1.reshape(1, D_HID))
    agg2 = _sc_agg16(src2d, dst2d, y2, zeros16)
    out = _tc3(agg2, y2, degp, b2.reshape(1, D_OUT))
    return out[:N]


# speedup vs baseline: 31.2715x; 1.6679x over previous
import functools

import jax
import jax.numpy as jnp
from jax import lax
from jax.experimental import pallas as pl
from jax.experimental.pallas import tpu as pltpu
from jax.experimental.pallas import tpu_sc as plsc

N = 10000
E = 320000
D_IN = 128
D_HID = 64
D_OUT = 16

NW = 32
CHUNK = 128
CHUNKS_PER_W = 79
EPAD = NW * CHUNK * CHUNKS_PER_W
NROWS = 10240
ROWS_PER_TILE = NROWS // 16
PAD_DST = N + 200
DEG_W = 8

_mesh = plsc.VectorSubcoreMesh(core_axis_name="c", subcore_axis_name="s")
_sc_params = pltpu.CompilerParams(
    use_tc_tiling_on_sc=False, needs_layout_passes=False)


def _wid():
    return lax.axis_index("s") * 2 + lax.axis_index("c")


@functools.partial(
    pl.kernel,
    out_type=jax.ShapeDtypeStruct((2, NROWS, DEG_W), jnp.float32),
    mesh=_mesh,
    compiler_params=_sc_params,
    scratch_types=[
        pltpu.VMEM((CHUNKS_PER_W, CHUNK), jnp.int32),
        pltpu.VMEM((CHUNK, DEG_W), jnp.float32),
        pltpu.VMEM_SHARED((NROWS, DEG_W), jnp.float32),
    ],
)
def _sc_degree(dst_hbm, ones_hbm, zeros_hbm, out_hbm, dst_v, ones_v, deg_sh):
    c = lax.axis_index("c")
    s = lax.axis_index("s")
    wid = _wid()
    pltpu.sync_copy(dst_hbm.at[pl.ds(wid * CHUNKS_PER_W, CHUNKS_PER_W)], dst_v)
    pltpu.sync_copy(ones_hbm, ones_v)
    base = s * ROWS_PER_TILE
    pltpu.sync_copy(zeros_hbm, deg_sh.at[pl.ds(base, ROWS_PER_TILE)])
    plsc.subcore_barrier()

    def body(j, carry):
        pltpu.sync_copy(ones_v, deg_sh.at[dst_v.at[j]], add=True)
        return carry

    lax.fori_loop(0, CHUNKS_PER_W, body, 0)
    plsc.subcore_barrier()
    pltpu.sync_copy(
        deg_sh.at[pl.ds(base, ROWS_PER_TILE)],
        out_hbm.at[c, pl.ds(base, ROWS_PER_TILE)],
    )


def _make_sc_agg(d):
    @functools.partial(
        pl.kernel,
        out_type=jax.ShapeDtypeStruct((2, NROWS, d), jnp.float32),
        mesh=_mesh,
        compiler_params=_sc_params,
        scratch_types=[
            pltpu.VMEM((CHUNKS_PER_W, CHUNK), jnp.int32),
            pltpu.VMEM((CHUNKS_PER_W, CHUNK), jnp.int32),
            pltpu.VMEM((2, CHUNK, d), jnp.float32),
            pltpu.VMEM_SHARED((NROWS, d), jnp.float32),
            pltpu.SemaphoreType.DMA,
            pltpu.SemaphoreType.DMA,
            pltpu.SemaphoreType.DMA,
            pltpu.SemaphoreType.DMA,
        ],
    )
    def sc_agg(src_hbm, dst_hbm, y_hbm, zeros_hbm, out_hbm,
               src_v, dst_v, rows_v, agg_sh, g0, g1, s0, s1):
        c = lax.axis_index("c")
        s = lax.axis_index("s")
        wid = _wid()
        base = s * ROWS_PER_TILE
        gsem = (g0, g1)
        ssem = (s0, s1)
        rb = wid * CHUNKS_PER_W
        pltpu.sync_copy(src_hbm.at[pl.ds(rb, CHUNKS_PER_W)], src_v)
        pltpu.sync_copy(dst_hbm.at[pl.ds(rb, CHUNKS_PER_W)], dst_v)
        pltpu.sync_copy(zeros_hbm, agg_sh.at[pl.ds(base, ROWS_PER_TILE)])
        plsc.subcore_barrier()

        def gather(j, b):
            pltpu.async_copy(y_hbm.at[src_v.at[j]], rows_v.at[b], gsem[b])

        gather(0, 0)

        def body(g2, carry):
            for b in range(2):
                j = g2 * 2 + b
                nb = 1 - b

                @pl.when(j >= 1)
                def _():
                    pltpu.make_async_copy(
                        y_hbm.at[pl.ds(0, CHUNK)], rows_v.at[nb],
                        ssem[nb]).wait()

                @pl.when(j + 1 < CHUNKS_PER_W)
                def _():
                    gather(j + 1, nb)

                pltpu.make_async_copy(
                    y_hbm.at[pl.ds(0, CHUNK)], rows_v.at[b], gsem[b]).wait()
                pltpu.async_copy(
                    rows_v.at[b], agg_sh.at[dst_v.at[j]], ssem[b], add=True)
            return carry

        lax.fori_loop(0, CHUNKS_PER_W // 2, body, 0)
        jlast = CHUNKS_PER_W - 1
        blast = jlast % 2
        pltpu.make_async_copy(
            y_hbm.at[pl.ds(0, CHUNK)], rows_v.at[blast], gsem[blast]).wait()
        pltpu.async_copy(
            rows_v.at[blast], agg_sh.at[dst_v.at[jlast]], ssem[blast],
            add=True)
        pltpu.make_async_copy(
            y_hbm.at[pl.ds(0, CHUNK)], rows_v.at[0], ssem[0]).wait()
        pltpu.make_async_copy(
            y_hbm.at[pl.ds(0, CHUNK)], rows_v.at[1], ssem[1]).wait()
        plsc.subcore_barrier()
        pltpu.sync_copy(
            agg_sh.at[pl.ds(base, ROWS_PER_TILE)],
            out_hbm.at[c, pl.ds(base, ROWS_PER_TILE)],
        )

    return sc_agg


_sc_agg64 = _make_sc_agg(D_HID)
_sc_agg16 = _make_sc_agg(D_OUT)


_R = 1024


def _dis_from(degp_ref):
    deg = degp_ref[0] + degp_ref[1] + 2.0
    return lax.rsqrt(deg)[:, 0:1]


def _tc1_body(x_ref, w1_ref, degp_ref, y1_ref):
    dis = _dis_from(degp_ref)
    xw = jnp.dot(x_ref[...], w1_ref[...], preferred_element_type=jnp.float32)
    y1_ref[...] = xw * dis


def _psum(aggp_ref):
    t = aggp_ref[0]
    for i in range(1, aggp_ref.shape[0]):
        t = t + aggp_ref[i]
    return t


def _tc2_body(aggp_ref, y1_ref, degp_ref, w2_ref, b1_ref, y2_ref):
    dis = _dis_from(degp_ref)
    pre = (_psum(aggp_ref) + 2.0 * y1_ref[...]) * dis + b1_ref[...]
    h = jnp.maximum(pre, 0.0)
    y2_ref[...] = jnp.dot(h, w2_ref[...], preferred_element_type=jnp.float32) * dis


def _tc3_body(aggp_ref, y2_ref, degp_ref, b2_ref, out_ref):
    dis = _dis_from(degp_ref)
    o = (_psum(aggp_ref) + 2.0 * y2_ref[...]) * dis + b2_ref[...]
    m = jnp.max(o, axis=1, keepdims=True)
    e = jnp.exp(o - m)
    lse = jnp.log(jnp.sum(e, axis=1, keepdims=True))
    out_ref[...] = o - m - lse


def _row_spec(d):
    return pl.BlockSpec((_R, d), lambda i: (i, 0))


def _part_spec(n, d):
    return pl.BlockSpec((n, _R, d), lambda i: (0, i, 0))


_deg_spec = pl.BlockSpec((2, _R, DEG_W), lambda i: (0, i, 0))


def _full_spec(a, b):
    return pl.BlockSpec((a, b), lambda i: (0, 0))


_GRID = (NROWS // _R,)

_tc1 = pl.pallas_call(
    _tc1_body,
    grid=_GRID,
    in_specs=[_row_spec(D_IN), _full_spec(D_IN, D_HID), _deg_spec],
    out_specs=_row_spec(D_HID),
    out_shape=jax.ShapeDtypeStruct((NROWS, D_HID), jnp.float32),
)

_tc2 = pl.pallas_call(
    _tc2_body,
    grid=_GRID,
    in_specs=[_part_spec(2, D_HID), _row_spec(D_HID), _deg_spec,
              _full_spec(D_HID, D_OUT), _full_spec(1, D_HID)],
    out_specs=_row_spec(D_OUT),
    out_shape=jax.ShapeDtypeStruct((NROWS, D_OUT), jnp.float32),
)

_tc3 = pl.pallas_call(
    _tc3_body,
    grid=_GRID,
    in_specs=[_part_spec(2, D_OUT), _row_spec(D_OUT), _deg_spec,
              _full_spec(1, D_OUT)],
    out_specs=_row_spec(D_OUT),
    out_shape=jax.ShapeDtypeStruct((NROWS, D_OUT), jnp.float32),
)


def kernel(x, edge_index, W1, b1, W2, b2):
    ei = edge_index.astype(jnp.int32)
    npad = EPAD - E
    src = jnp.concatenate([ei[0], jnp.full((npad,), N, jnp.int32)])
    dst = jnp.concatenate([ei[1], jnp.full((npad,), PAD_DST, jnp.int32)])
    src2d = src.reshape(EPAD // CHUNK, CHUNK)
    dst2d = dst.reshape(EPAD // CHUNK, CHUNK)

    x_pad = jnp.pad(x, ((0, NROWS - N), (0, 0)))
    ones_deg = jnp.ones((CHUNK, DEG_W), jnp.float32)
    zeros_deg = jnp.zeros((ROWS_PER_TILE, DEG_W), jnp.float32)
    degp = _sc_degree(dst2d, ones_deg, zeros_deg)
    y1 = _tc1(x_pad, W1, degp)
    zeros64 = jnp.zeros((ROWS_PER_TILE, D_HID), jnp.float32)
    zeros16 = jnp.zeros((ROWS_PER_TILE, D_OUT), jnp.float32)
    agg1 = _sc_agg64(src2d, dst2d, y1, zeros64)
    y2 = _tc2(agg1, y1, degp, W2, b1.reshape(1, D_HID))
    agg2 = _sc_agg16(src2d, dst2d, y2, zeros16)
    out = _tc3(agg2, y2, degp, b2.reshape(1, D_OUT))
    return out[:N]


# speedup vs baseline: 45.7687x; 1.4636x over previous
import functools

import jax
import jax.numpy as jnp
from jax import lax
from jax.experimental import pallas as pl
from jax.experimental.pallas import tpu as pltpu
from jax.experimental.pallas import tpu_sc as plsc

N = 10000
E = 320000
D_IN = 128
D_HID = 64
D_OUT = 16

NW = 32
CHUNK = 128
NCH = E // CHUNK
RPW = NCH // NW
NEXTRA = NCH - NW * RPW
BUFR = 3
NITER = RPW // BUFR
NROWS = 10240
ROWS_PER_TILE = NROWS // 16
DEG_W = 4

_mesh = plsc.VectorSubcoreMesh(core_axis_name="c", subcore_axis_name="s")
_sc_params = pltpu.CompilerParams(
    use_tc_tiling_on_sc=False, needs_layout_passes=False)


def _wid():
    return lax.axis_index("s") * 2 + lax.axis_index("c")


@functools.partial(
    pl.kernel,
    out_type=jax.ShapeDtypeStruct((2, NROWS, DEG_W), jnp.float32),
    mesh=_mesh,
    compiler_params=_sc_params,
    scratch_types=[
        pltpu.VMEM((RPW + 1, CHUNK), jnp.int32),
        pltpu.VMEM((CHUNK, DEG_W), jnp.float32),
        pltpu.VMEM_SHARED((NROWS, DEG_W), jnp.float32),
    ],
)
def _sc_degree(dst_hbm, ones_hbm, zeros_hbm, out_hbm, dst_v, ones_v, deg_sh):
    c = lax.axis_index("c")
    s = lax.axis_index("s")
    wid = _wid()
    pltpu.sync_copy(dst_hbm.at[pl.ds(wid * RPW, RPW)],
                    dst_v.at[pl.ds(0, RPW)])

    @pl.when(wid < NEXTRA)
    def _():
        pltpu.sync_copy(dst_hbm.at[pl.ds(NW * RPW + wid, 1)],
                        dst_v.at[pl.ds(RPW, 1)])

    pltpu.sync_copy(ones_hbm, ones_v)
    base = s * ROWS_PER_TILE
    pltpu.sync_copy(zeros_hbm, deg_sh.at[pl.ds(base, ROWS_PER_TILE)])
    plsc.subcore_barrier()

    def body(j, carry):
        pltpu.sync_copy(ones_v, deg_sh.at[dst_v.at[j]], add=True)
        return carry

    lax.fori_loop(0, RPW, body, 0)

    @pl.when(wid < NEXTRA)
    def _():
        pltpu.sync_copy(ones_v, deg_sh.at[dst_v.at[RPW]], add=True)

    plsc.subcore_barrier()
    pltpu.sync_copy(
        deg_sh.at[pl.ds(base, ROWS_PER_TILE)],
        out_hbm.at[c, pl.ds(base, ROWS_PER_TILE)],
    )


def _make_sc_agg(d):
    @functools.partial(
        pl.kernel,
        out_type=jax.ShapeDtypeStruct((2, NROWS, d), jnp.float32),
        mesh=_mesh,
        compiler_params=_sc_params,
        scratch_types=[
            pltpu.VMEM((RPW + 1, CHUNK), jnp.int32),
            pltpu.VMEM((RPW + 1, CHUNK), jnp.int32),
            pltpu.VMEM((2, BUFR * CHUNK, d), jnp.float32),
            pltpu.VMEM_SHARED((NROWS, d), jnp.float32),
            pltpu.SemaphoreType.DMA,
            pltpu.SemaphoreType.DMA,
            pltpu.SemaphoreType.DMA,
            pltpu.SemaphoreType.DMA,
        ],
    )
    def sc_agg(src_hbm, dst_hbm, y_hbm, zeros_hbm, out_hbm,
               src_v, dst_v, rows_v, agg_sh, g0, g1, s0, s1):
        c = lax.axis_index("c")
        s = lax.axis_index("s")
        wid = _wid()
        base = s * ROWS_PER_TILE
        gsem = (g0, g1)
        ssem = (s0, s1)
        rb = wid * RPW
        pltpu.sync_copy(src_hbm.at[pl.ds(rb, RPW)], src_v.at[pl.ds(0, RPW)])
        pltpu.sync_copy(dst_hbm.at[pl.ds(rb, RPW)], dst_v.at[pl.ds(0, RPW)])

        @pl.when(wid < NEXTRA)
        def _():
            pltpu.sync_copy(src_hbm.at[pl.ds(NW * RPW + wid, 1)],
                            src_v.at[pl.ds(RPW, 1)])
            pltpu.sync_copy(dst_hbm.at[pl.ds(NW * RPW + wid, 1)],
                            dst_v.at[pl.ds(RPW, 1)])

        def gather(j, b):
            for k in range(BUFR):
                pltpu.async_copy(y_hbm.at[src_v.at[j * BUFR + k]],
                                 rows_v.at[b, pl.ds(k * CHUNK, CHUNK)],
                                 gsem[b])

        def scatter(j, b):
            for k in range(BUFR):
                pltpu.async_copy(rows_v.at[b, pl.ds(k * CHUNK, CHUNK)],
                                 agg_sh.at[dst_v.at[j * BUFR + k]],
                                 ssem[b], add=True)

        gather(0, 0)
        pltpu.sync_copy(zeros_hbm, agg_sh.at[pl.ds(base, ROWS_PER_TILE)])
        plsc.subcore_barrier()

        def body(g2, carry):
            for b in range(2):
                j = g2 * 2 + b
                nb = 1 - b

                @pl.when(j >= 1)
                def _():
                    pltpu.make_async_copy(
                        y_hbm.at[pl.ds(0, BUFR * CHUNK)], rows_v.at[nb],
                        ssem[nb]).wait()

                @pl.when(j + 1 < NITER)
                def _():
                    gather(j + 1, nb)

                pltpu.make_async_copy(
                    y_hbm.at[pl.ds(0, BUFR * CHUNK)], rows_v.at[b],
                    gsem[b]).wait()
                scatter(j, b)
            return carry

        lax.fori_loop(0, NITER // 2, body, 0)
        pltpu.make_async_copy(
            y_hbm.at[pl.ds(0, BUFR * CHUNK)], rows_v.at[1], ssem[1]).wait()

        @pl.when(wid < NEXTRA)
        def _():
            pltpu.async_copy(y_hbm.at[src_v.at[RPW]],
                             rows_v.at[0, pl.ds(0, CHUNK)], g0)
            pltpu.make_async_copy(
                y_hbm.at[pl.ds(0, CHUNK)],
                rows_v.at[0, pl.ds(0, CHUNK)], g0).wait()
            pltpu.async_copy(rows_v.at[0, pl.ds(0, CHUNK)],
                             agg_sh.at[dst_v.at[RPW]], s0, add=True)
            pltpu.make_async_copy(
                y_hbm.at[pl.ds(0, CHUNK)],
                rows_v.at[0, pl.ds(0, CHUNK)], s0).wait()

        plsc.subcore_barrier()
        pltpu.sync_copy(
            agg_sh.at[pl.ds(base, ROWS_PER_TILE)],
            out_hbm.at[c, pl.ds(base, ROWS_PER_TILE)],
        )

    return sc_agg


_sc_agg64 = _make_sc_agg(D_HID)
_sc_agg16 = _make_sc_agg(D_OUT)


_R = 1024


def _dis_from(degp_ref):
    deg = degp_ref[0] + degp_ref[1] + 2.0
    return lax.rsqrt(deg)[:, 0:1]


def _tc1_body(x_ref, w1_ref, degp_ref, y1_ref):
    dis = _dis_from(degp_ref)
    xw = jnp.dot(x_ref[...], w1_ref[...], preferred_element_type=jnp.float32)
    y1_ref[...] = xw * dis


def _psum(aggp_ref):
    t = aggp_ref[0]
    for i in range(1, aggp_ref.shape[0]):
        t = t + aggp_ref[i]
    return t


def _tc2_body(aggp_ref, y1_ref, degp_ref, w2_ref, b1_ref, y2_ref):
    dis = _dis_from(degp_ref)
    pre = (_psum(aggp_ref) + 2.0 * y1_ref[...]) * dis + b1_ref[...]
    h = jnp.maximum(pre, 0.0)
    y2_ref[...] = jnp.dot(h, w2_ref[...], preferred_element_type=jnp.float32) * dis


def _tc3_body(aggp_ref, y2_ref, degp_ref, b2_ref, out_ref):
    dis = _dis_from(degp_ref)
    o = (_psum(aggp_ref) + 2.0 * y2_ref[...]) * dis + b2_ref[...]
    m = jnp.max(o, axis=1, keepdims=True)
    e = jnp.exp(o - m)
    lse = jnp.log(jnp.sum(e, axis=1, keepdims=True))
    out_ref[...] = o - m - lse


def _row_spec(d):
    return pl.BlockSpec((_R, d), lambda i: (i, 0))


def _part_spec(n, d):
    return pl.BlockSpec((n, _R, d), lambda i: (0, i, 0))


_deg_spec = pl.BlockSpec((2, _R, DEG_W), lambda i: (0, i, 0))


def _full_spec(a, b):
    return pl.BlockSpec((a, b), lambda i: (0, 0))


_GRID = (NROWS // _R,)

_tc1 = pl.pallas_call(
    _tc1_body,
    grid=_GRID,
    in_specs=[_row_spec(D_IN), _full_spec(D_IN, D_HID), _deg_spec],
    out_specs=_row_spec(D_HID),
    out_shape=jax.ShapeDtypeStruct((NROWS, D_HID), jnp.float32),
)

_tc2 = pl.pallas_call(
    _tc2_body,
    grid=_GRID,
    in_specs=[_part_spec(2, D_HID), _row_spec(D_HID), _deg_spec,
              _full_spec(D_HID, D_OUT), _full_spec(1, D_HID)],
    out_specs=_row_spec(D_OUT),
    out_shape=jax.ShapeDtypeStruct((NROWS, D_OUT), jnp.float32),
)

_tc3 = pl.pallas_call(
    _tc3_body,
    grid=_GRID,
    in_specs=[_part_spec(2, D_OUT), _row_spec(D_OUT), _deg_spec,
              _full_spec(1, D_OUT)],
    out_specs=_row_spec(D_OUT),
    out_shape=jax.ShapeDtypeStruct((NROWS, D_OUT), jnp.float32),
)


def kernel(x, edge_index, W1, b1, W2, b2):
    ei = edge_index.astype(jnp.int32)
    src2d = ei[0].reshape(NCH, CHUNK)
    dst2d = ei[1].reshape(NCH, CHUNK)

    x_pad = jnp.pad(x, ((0, NROWS - N), (0, 0)))
    ones_deg = jnp.ones((CHUNK, DEG_W), jnp.float32)
    zeros_deg = jnp.zeros((ROWS_PER_TILE, DEG_W), jnp.float32)
    degp = _sc_degree(dst2d, ones_deg, zeros_deg)
    y1 = _tc1(x_pad, W1, degp)
    zeros64 = jnp.zeros((ROWS_PER_TILE, D_HID), jnp.float32)
    zeros16 = jnp.zeros((ROWS_PER_TILE, D_OUT), jnp.float32)
    agg1 = _sc_agg64(src2d, dst2d, y1, zeros64)
    y2 = _tc2(agg1, y1, degp, W2, b1.reshape(1, D_HID))
    agg2 = _sc_agg16(src2d, dst2d, y2, zeros16)
    out = _tc3(agg2, y2, degp, b2.reshape(1, D_OUT))
    return out[:N]


# speedup vs baseline: 46.9467x; 1.0257x over previous
import functools

import jax
import jax.numpy as jnp
from jax import lax
from jax.experimental import pallas as pl
from jax.experimental.pallas import tpu as pltpu
from jax.experimental.pallas import tpu_sc as plsc

N = 10000
E = 320000
D_IN = 128
D_HID = 64
D_OUT = 16

NW = 32
CHUNK = 128
NCH = E // CHUNK
RPW = NCH // NW
NEXTRA = NCH - NW * RPW
BUFR = 3
NITER = RPW // BUFR
NROWS = 10240
ROWS_PER_TILE = NROWS // 16
DEG_W = 4

_mesh = plsc.VectorSubcoreMesh(core_axis_name="c", subcore_axis_name="s")
_sc_params = pltpu.CompilerParams(
    use_tc_tiling_on_sc=False, needs_layout_passes=False)


def _wid():
    return lax.axis_index("s") * 2 + lax.axis_index("c")


@functools.partial(
    pl.kernel,
    out_type=jax.ShapeDtypeStruct((2, NROWS, DEG_W), jnp.float32),
    mesh=_mesh,
    compiler_params=_sc_params,
    scratch_types=[
        pltpu.VMEM((RPW + 1, CHUNK), jnp.int32),
        pltpu.VMEM((CHUNK, DEG_W), jnp.float32),
        pltpu.VMEM_SHARED((NROWS, DEG_W), jnp.float32),
    ],
)
def _sc_degree(dst_hbm, ones_hbm, zeros_hbm, out_hbm, dst_v, ones_v, deg_sh):
    c = lax.axis_index("c")
    s = lax.axis_index("s")
    wid = _wid()
    pltpu.sync_copy(dst_hbm.at[pl.ds(wid * RPW, RPW)],
                    dst_v.at[pl.ds(0, RPW)])

    @pl.when(wid < NEXTRA)
    def _():
        pltpu.sync_copy(dst_hbm.at[pl.ds(NW * RPW + wid, 1)],
                        dst_v.at[pl.ds(RPW, 1)])

    pltpu.sync_copy(ones_hbm, ones_v)
    base = s * ROWS_PER_TILE
    pltpu.sync_copy(zeros_hbm, deg_sh.at[pl.ds(base, ROWS_PER_TILE)])
    plsc.subcore_barrier()

    def body(j, carry):
        pltpu.sync_copy(ones_v, deg_sh.at[dst_v.at[j]], add=True)
        return carry

    lax.fori_loop(0, RPW, body, 0)

    @pl.when(wid < NEXTRA)
    def _():
        pltpu.sync_copy(ones_v, deg_sh.at[dst_v.at[RPW]], add=True)

    plsc.subcore_barrier()
    pltpu.sync_copy(
        deg_sh.at[pl.ds(base, ROWS_PER_TILE)],
        out_hbm.at[c, pl.ds(base, ROWS_PER_TILE)],
    )


def _make_sc_agg(d):
    @functools.partial(
        pl.kernel,
        out_type=jax.ShapeDtypeStruct((2, NROWS, d), jnp.float32),
        mesh=_mesh,
        compiler_params=_sc_params,
        scratch_types=[
            pltpu.VMEM((RPW + 1, CHUNK), jnp.int32),
            pltpu.VMEM((RPW + 1, CHUNK), jnp.int32),
            pltpu.VMEM((2, BUFR * CHUNK, d), jnp.float32),
            pltpu.VMEM_SHARED((NROWS, d), jnp.float32),
            pltpu.SemaphoreType.DMA,
            pltpu.SemaphoreType.DMA,
            pltpu.SemaphoreType.DMA,
            pltpu.SemaphoreType.DMA,
        ],
    )
    def sc_agg(src_hbm, dst_hbm, y_hbm, zeros_hbm, out_hbm,
               src_v, dst_v, rows_v, agg_sh, g0, g1, s0, s1):
        c = lax.axis_index("c")
        s = lax.axis_index("s")
        wid = _wid()
        base = s * ROWS_PER_TILE
        gsem = (g0, g1)
        ssem = (s0, s1)
        rb = wid * RPW
        pltpu.sync_copy(src_hbm.at[pl.ds(rb, RPW)], src_v.at[pl.ds(0, RPW)])
        pltpu.sync_copy(dst_hbm.at[pl.ds(rb, RPW)], dst_v.at[pl.ds(0, RPW)])

        @pl.when(wid < NEXTRA)
        def _():
            pltpu.sync_copy(src_hbm.at[pl.ds(NW * RPW + wid, 1)],
                            src_v.at[pl.ds(RPW, 1)])
            pltpu.sync_copy(dst_hbm.at[pl.ds(NW * RPW + wid, 1)],
                            dst_v.at[pl.ds(RPW, 1)])

        def gather(j, b):
            for k in range(BUFR):
                pltpu.async_copy(y_hbm.at[src_v.at[j * BUFR + k]],
                                 rows_v.at[b, pl.ds(k * CHUNK, CHUNK)],
                                 gsem[b])

        def scatter(j, b):
            for k in range(BUFR):
                pltpu.async_copy(rows_v.at[b, pl.ds(k * CHUNK, CHUNK)],
                                 agg_sh.at[dst_v.at[j * BUFR + k]],
                                 ssem[b], add=True)

        gather(0, 0)
        pltpu.sync_copy(zeros_hbm, agg_sh.at[pl.ds(base, ROWS_PER_TILE)])
        plsc.subcore_barrier()

        def body(g2, carry):
            for b in range(2):
                j = g2 * 2 + b
                nb = 1 - b

                @pl.when(j >= 1)
                def _():
                    pltpu.make_async_copy(
                        y_hbm.at[pl.ds(0, BUFR * CHUNK)], rows_v.at[nb],
                        ssem[nb]).wait()

                @pl.when(j + 1 < NITER)
                def _():
                    gather(j + 1, nb)

                pltpu.make_async_copy(
                    y_hbm.at[pl.ds(0, BUFR * CHUNK)], rows_v.at[b],
                    gsem[b]).wait()
                scatter(j, b)
            return carry

        lax.fori_loop(0, NITER // 2, body, 0)
        pltpu.make_async_copy(
            y_hbm.at[pl.ds(0, BUFR * CHUNK)], rows_v.at[1], ssem[1]).wait()

        @pl.when(wid < NEXTRA)
        def _():
            pltpu.async_copy(y_hbm.at[src_v.at[RPW]],
                             rows_v.at[0, pl.ds(0, CHUNK)], g0)
            pltpu.make_async_copy(
                y_hbm.at[pl.ds(0, CHUNK)],
                rows_v.at[0, pl.ds(0, CHUNK)], g0).wait()
            pltpu.async_copy(rows_v.at[0, pl.ds(0, CHUNK)],
                             agg_sh.at[dst_v.at[RPW]], s0, add=True)
            pltpu.make_async_copy(
                y_hbm.at[pl.ds(0, CHUNK)],
                rows_v.at[0, pl.ds(0, CHUNK)], s0).wait()

        plsc.subcore_barrier()
        pltpu.sync_copy(
            agg_sh.at[pl.ds(base, ROWS_PER_TILE)],
            out_hbm.at[c, pl.ds(base, ROWS_PER_TILE)],
        )

    return sc_agg


_sc_agg64 = _make_sc_agg(D_HID)
_sc_agg16 = _make_sc_agg(D_OUT)


def _dis_from(degp_ref):
    deg = degp_ref[0] + degp_ref[1] + 2.0
    return lax.rsqrt(deg)[:N, 0:1]


def _psum10k(aggp_ref):
    return aggp_ref[0, :N] + aggp_ref[1, :N]


def _tc1_body(x_ref, w1_ref, degp_ref, y1_ref):
    dis = _dis_from(degp_ref)
    xw = jnp.dot(x_ref[...], w1_ref[...], preferred_element_type=jnp.float32)
    y1_ref[...] = xw * dis


def _tc2_body(aggp_ref, y1_ref, degp_ref, w2_ref, b1_ref, y2_ref):
    dis = _dis_from(degp_ref)
    pre = (_psum10k(aggp_ref) + 2.0 * y1_ref[...]) * dis + b1_ref[...]
    h = jnp.maximum(pre, 0.0)
    y2_ref[...] = jnp.dot(h, w2_ref[...], preferred_element_type=jnp.float32) * dis


def _tc3_body(aggp_ref, y2_ref, degp_ref, b2_ref, out_ref):
    dis = _dis_from(degp_ref)
    o = (_psum10k(aggp_ref) + 2.0 * y2_ref[...]) * dis + b2_ref[...]
    m = jnp.max(o, axis=1, keepdims=True)
    e = jnp.exp(o - m)
    lse = jnp.log(jnp.sum(e, axis=1, keepdims=True))
    out_ref[...] = o - m - lse


_tc1 = pl.pallas_call(
    _tc1_body,
    out_shape=jax.ShapeDtypeStruct((N, D_HID), jnp.float32),
)

_tc2 = pl.pallas_call(
    _tc2_body,
    out_shape=jax.ShapeDtypeStruct((N, D_OUT), jnp.float32),
)

_tc3 = pl.pallas_call(
    _tc3_body,
    out_shape=jax.ShapeDtypeStruct((N, D_OUT), jnp.float32),
)


def kernel(x, edge_index, W1, b1, W2, b2):
    ei = edge_index.astype(jnp.int32)
    src2d = ei[0].reshape(NCH, CHUNK)
    dst2d = ei[1].reshape(NCH, CHUNK)

    ones_deg = jnp.ones((CHUNK, DEG_W), jnp.float32)
    zeros_deg = jnp.zeros((ROWS_PER_TILE, DEG_W), jnp.float32)
    zeros64 = jnp.zeros((ROWS_PER_TILE, D_HID), jnp.float32)
    zeros16 = jnp.zeros((ROWS_PER_TILE, D_OUT), jnp.float32)

    degp = _sc_degree(dst2d, ones_deg, zeros_deg)
    y1 = _tc1(x, W1, degp)
    agg1 = _sc_agg64(src2d, dst2d, y1, zeros64)
    y2 = _tc2(agg1, y1, degp, W2, b1.reshape(1, D_HID))
    agg2 = _sc_agg16(src2d, dst2d, y2, zeros16)
    return _tc3(agg2, y2, degp, b2.reshape(1, D_OUT))
